# Initial kernel scaffold; baseline (speedup 1.0000x reference)
import functools

import jax
import jax.numpy as jnp
from jax import lax
from jax.experimental import pallas as pl
from jax.experimental.pallas import tpu as pltpu
from jax.experimental.pallas import tpu_sc as plsc

N = 10000
E = 320000
D = 128

NC = 2
NS = 16
NW = NC * NS

CH = 128
NCHUNK = 80
EPT = CH * NCHUNK
E_PAD = EPT * NW
ACC_ROWS = 10240
ZROWS = ACC_ROWS // NS


def _sc_agg_body(x_hbm, src_hbm, dst_hbm, out_hbm,
                 src_v, dst_v, sbuf, dbuf, rows, zbuf, acc, gsem):
  core = lax.axis_index("c")
  sid = lax.axis_index("s")
  tid = sid * NC + core

  def _zero_body(k, _):
    r = k // 8
    c = (k % 8) * 16
    zbuf[r, pl.ds(c, 16)] = jnp.zeros((16,), jnp.float32)
    return 0
  lax.fori_loop(0, CH * 8, _zero_body, 0)
  for z in range(ZROWS // CH):
    pltpu.sync_copy(zbuf, acc.at[pl.ds(sid * ZROWS + z * CH, CH)])
  plsc.subcore_barrier()

  base = tid * EPT
  pltpu.sync_copy(src_hbm.at[pl.ds(base, EPT)], src_v)
  pltpu.sync_copy(dst_hbm.at[pl.ds(base, EPT)], dst_v)

  def _chunk_body(j, _):
    off = j * CH
    pltpu.sync_copy(src_v.at[pl.ds(off, CH)], sbuf)
    pltpu.sync_copy(dst_v.at[pl.ds(off, CH)], dbuf)
    pltpu.async_copy(x_hbm.at[sbuf], rows, gsem).wait()
    pltpu.sync_copy(rows, acc.at[dbuf], add=True)
    return 0
  lax.fori_loop(0, NCHUNK, _chunk_body, 0)

  plsc.subcore_barrier()
  rpt = N // NS
  pltpu.sync_copy(acc.at[pl.ds(sid * rpt, rpt)],
                  out_hbm.at[pl.ds(core * N + sid * rpt, rpt)])


_sc_agg = functools.partial(
    pl.kernel,
    out_type=jax.ShapeDtypeStruct((NC * N, D), jnp.float32),
    mesh=plsc.VectorSubcoreMesh(
        core_axis_name="c", subcore_axis_name="s", num_cores=NC,
        num_subcores=NS),
    scratch_types=[
        pltpu.VMEM((EPT,), jnp.int32),
        pltpu.VMEM((EPT,), jnp.int32),
        pltpu.VMEM((CH,), jnp.int32),
        pltpu.VMEM((CH,), jnp.int32),
        pltpu.VMEM((CH, D), jnp.float32),
        pltpu.VMEM((CH, D), jnp.float32),
        pltpu.VMEM_SHARED((ACC_ROWS, D), jnp.float32),
        pltpu.SemaphoreType.DMA,
    ],
)(_sc_agg_body)


def _agg(h, src_p, dst_p):
  return _sc_agg(h, src_p, dst_p)


ROWS_BLK = 1000
GRID = N // ROWS_BLK


def _mlp_body(relu_out, x_ref, a0_ref, a1_ref, w1_ref, b1_ref, w2_ref,
              b2_ref, o_ref):
  h = x_ref[...] + a0_ref[...] + a1_ref[...]
  h = jnp.maximum(
      jnp.dot(h, w1_ref[...], preferred_element_type=jnp.float32)
      + b1_ref[...], 0.0)
  h = jnp.dot(h, w2_ref[...], preferred_element_type=jnp.float32) + b2_ref[...]
  if relu_out:
    h = jnp.maximum(h, 0.0)
  o_ref[...] = h


def _row_spec(shift=0):
  return pl.BlockSpec((ROWS_BLK, D), lambda i: (i + shift, 0))


def _w_spec():
  return pl.BlockSpec((D, D), lambda i: (0, 0))


def _b_spec():
  return pl.BlockSpec((1, D), lambda i: (0, 0))


def _mlp(x, agg2, w1, b1, w2, b2, relu_out):
  return pl.pallas_call(
      functools.partial(_mlp_body, relu_out),
      grid=(GRID,),
      in_specs=[_row_spec(), _row_spec(), _row_spec(GRID),
                _w_spec(), _b_spec(), _w_spec(), _b_spec()],
      out_specs=_row_spec(),
      out_shape=jax.ShapeDtypeStruct((N, D), jnp.float32),
  )(x, agg2, agg2, w1, b1.reshape(1, D), w2, b2.reshape(1, D))


def _head_body(x_ref, a0_ref, a1_ref, w1_ref, b1_ref, w2_ref, b2_ref,
               l1w_ref, l1b_ref, l2w_ref, l2b_ref, o_ref):
  h = x_ref[...] + a0_ref[...] + a1_ref[...]
  h = jnp.maximum(
      jnp.dot(h, w1_ref[...], preferred_element_type=jnp.float32)
      + b1_ref[...], 0.0)
  h = jnp.dot(h, w2_ref[...], preferred_element_type=jnp.float32) + b2_ref[...]
  h = jnp.maximum(
      jnp.dot(h, l1w_ref[...], preferred_element_type=jnp.float32)
      + l1b_ref[...], 0.0)
  z = jnp.dot(h, l2w_ref[...], preferred_element_type=jnp.float32) + l2b_ref[...]
  m = jnp.max(z, axis=1, keepdims=True)
  e = z - m
  o_ref[...] = e - jnp.log(jnp.sum(jnp.exp(e), axis=1, keepdims=True))


def _head(x, agg2, w1, b1, w2, b2, l1w, l1b, l2w, l2b):
  return pl.pallas_call(
      _head_body,
      grid=(GRID,),
      in_specs=[_row_spec(), _row_spec(), _row_spec(GRID),
                _w_spec(), _b_spec(), _w_spec(), _b_spec(),
                _w_spec(), _b_spec(), _w_spec(), _b_spec()],
      out_specs=_row_spec(),
      out_shape=jax.ShapeDtypeStruct((N, D), jnp.float32),
  )(x, agg2, agg2, w1, b1.reshape(1, D), w2, b2.reshape(1, D),
    l1w, l1b.reshape(1, D), l2w, l2b.reshape(1, D))


def kernel(x, edge_index, batch, pool,
           c1_W1, c1_b1, c1_W2, c1_b2,
           c2_W1, c2_b1, c2_W2, c2_b2,
           c3_W1, c3_b1, c3_W2, c3_b2,
           lin1_W, lin1_b, lin2_W, lin2_b):
  src = edge_index[0]
  dst = edge_index[1]
  pad = E_PAD - E
  src_p = jnp.concatenate([src, jnp.zeros((pad,), jnp.int32)])
  dst_p = jnp.concatenate([dst, jnp.full((pad,), N, jnp.int32)])

  a = _agg(x, src_p, dst_p)
  h = _mlp(x, a, c1_W1, c1_b1, c1_W2, c1_b2, relu_out=True)
  a = _agg(h, src_p, dst_p)
  h = _mlp(h, a, c2_W1, c2_b1, c2_W2, c2_b2, relu_out=True)
  a = _agg(h, src_p, dst_p)
  return _head(h, a, c3_W1, c3_b1, c3_W2, c3_b2,
               lin1_W, lin1_b, lin2_W, lin2_b)


# speedup vs baseline: 2.5759x; 2.5759x over previous
import functools

import jax
import jax.numpy as jnp
from jax import lax
from jax.experimental import pallas as pl
from jax.experimental.pallas import tpu as pltpu
from jax.experimental.pallas import tpu_sc as plsc

N = 10000
E = 320000
D = 128

NC = 2
NS = 16
NW = NC * NS

CH = 128
NCHUNK = 80
EPT = CH * NCHUNK
E_PAD = EPT * NW
ACC_ROWS = 10240
ZROWS = ACC_ROWS // NS


def _sc_agg_body(x_hbm, src_hbm, dst_hbm, out_hbm,
                 src_v, dst_v, rows, acc, gsem):
  core = lax.axis_index("c")
  sid = lax.axis_index("s")
  tid = sid * NC + core

  def _zero_body(k, _):
    r = k // 8
    c = (k % 8) * 16
    rows[r, pl.ds(c, 16)] = jnp.zeros((16,), jnp.float32)
    return 0
  lax.fori_loop(0, CH * 8, _zero_body, 0)
  for z in range(ZROWS // CH):
    pltpu.sync_copy(rows, acc.at[pl.ds(sid * ZROWS + z * CH, CH)])
  plsc.subcore_barrier()

  base = tid * NCHUNK
  pltpu.sync_copy(src_hbm.at[pl.ds(base, NCHUNK)], src_v)
  pltpu.sync_copy(dst_hbm.at[pl.ds(base, NCHUNK)], dst_v)

  def _chunk_body(j, _):
    pltpu.async_copy(x_hbm.at[src_v.at[j]], rows, gsem).wait()
    pltpu.sync_copy(rows, acc.at[dst_v.at[j]], add=True)
    return 0
  lax.fori_loop(0, NCHUNK, _chunk_body, 0)

  plsc.subcore_barrier()
  rpt = 624
  pltpu.sync_copy(acc.at[pl.ds(sid * rpt, rpt)],
                  out_hbm.at[pl.ds(core * N + sid * rpt, rpt)])

  @pl.when(sid == NS - 1)
  def _tail():
    tail0 = NS * rpt
    pltpu.sync_copy(acc.at[pl.ds(tail0, N - tail0)],
                    out_hbm.at[pl.ds(core * N + tail0, N - tail0)])


@functools.cache
def _sc_agg():
  return functools.partial(
      pl.kernel,
      out_type=jax.ShapeDtypeStruct((NC * N, D), jnp.float32),
      mesh=plsc.VectorSubcoreMesh(
          core_axis_name="c", subcore_axis_name="s", num_cores=NC,
          num_subcores=NS),
      scratch_types=[
          pltpu.VMEM((NCHUNK, CH), jnp.int32),
          pltpu.VMEM((NCHUNK, CH), jnp.int32),
          pltpu.VMEM((CH, D), jnp.float32),
          pltpu.VMEM_SHARED((ACC_ROWS, D), jnp.float32),
          pltpu.SemaphoreType.DMA,
      ],
  )(_sc_agg_body)


def _agg(h, src_p, dst_p):
  return _sc_agg()(h, src_p, dst_p)


ROWS_BLK = 1000
GRID = N // ROWS_BLK


def _mlp_body(relu_out, x_ref, a0_ref, a1_ref, w1_ref, b1_ref, w2_ref,
              b2_ref, o_ref):
  h = x_ref[...] + a0_ref[...] + a1_ref[...]
  h = jnp.maximum(
      jnp.dot(h, w1_ref[...], preferred_element_type=jnp.float32)
      + b1_ref[...], 0.0)
  h = jnp.dot(h, w2_ref[...], preferred_element_type=jnp.float32) + b2_ref[...]
  if relu_out:
    h = jnp.maximum(h, 0.0)
  o_ref[...] = h


def _row_spec(shift=0):
  return pl.BlockSpec((ROWS_BLK, D), lambda i: (i + shift, 0))


def _w_spec():
  return pl.BlockSpec((D, D), lambda i: (0, 0))


def _b_spec():
  return pl.BlockSpec((1, D), lambda i: (0, 0))


def _mlp(x, agg2, w1, b1, w2, b2, relu_out):
  return pl.pallas_call(
      functools.partial(_mlp_body, relu_out),
      grid=(GRID,),
      in_specs=[_row_spec(), _row_spec(), _row_spec(GRID),
                _w_spec(), _b_spec(), _w_spec(), _b_spec()],
      out_specs=_row_spec(),
      out_shape=jax.ShapeDtypeStruct((N, D), jnp.float32),
  )(x, agg2, agg2, w1, b1.reshape(1, D), w2, b2.reshape(1, D))


def _head_body(x_ref, a0_ref, a1_ref, w1_ref, b1_ref, w2_ref, b2_ref,
               l1w_ref, l1b_ref, l2w_ref, l2b_ref, o_ref):
  h = x_ref[...] + a0_ref[...] + a1_ref[...]
  h = jnp.maximum(
      jnp.dot(h, w1_ref[...], preferred_element_type=jnp.float32)
      + b1_ref[...], 0.0)
  h = jnp.dot(h, w2_ref[...], preferred_element_type=jnp.float32) + b2_ref[...]
  h = jnp.maximum(
      jnp.dot(h, l1w_ref[...], preferred_element_type=jnp.float32)
      + l1b_ref[...], 0.0)
  z = jnp.dot(h, l2w_ref[...], preferred_element_type=jnp.float32) + l2b_ref[...]
  m = jnp.max(z, axis=1, keepdims=True)
  e = z - m
  o_ref[...] = e - jnp.log(jnp.sum(jnp.exp(e), axis=1, keepdims=True))


def _head(x, agg2, w1, b1, w2, b2, l1w, l1b, l2w, l2b):
  return pl.pallas_call(
      _head_body,
      grid=(GRID,),
      in_specs=[_row_spec(), _row_spec(), _row_spec(GRID),
                _w_spec(), _b_spec(), _w_spec(), _b_spec(),
                _w_spec(), _b_spec(), _w_spec(), _b_spec()],
      out_specs=_row_spec(),
      out_shape=jax.ShapeDtypeStruct((N, D), jnp.float32),
  )(x, agg2, agg2, w1, b1.reshape(1, D), w2, b2.reshape(1, D),
    l1w, l1b.reshape(1, D), l2w, l2b.reshape(1, D))


def kernel(x, edge_index, batch, pool,
           c1_W1, c1_b1, c1_W2, c1_b2,
           c2_W1, c2_b1, c2_W2, c2_b2,
           c3_W1, c3_b1, c3_W2, c3_b2,
           lin1_W, lin1_b, lin2_W, lin2_b):
  src = edge_index[0]
  dst = edge_index[1]
  pad = E_PAD - E
  src_p = jnp.concatenate([src, jnp.zeros((pad,), jnp.int32)])
  dst_p = jnp.concatenate([dst, jnp.full((pad,), N, jnp.int32)])
  src_p = src_p.reshape(NW * NCHUNK, CH)
  dst_p = dst_p.reshape(NW * NCHUNK, CH)

  a = _agg(x, src_p, dst_p)
  h = _mlp(x, a, c1_W1, c1_b1, c1_W2, c1_b2, relu_out=True)
  a = _agg(h, src_p, dst_p)
  h = _mlp(h, a, c2_W1, c2_b1, c2_W2, c2_b2, relu_out=True)
  a = _agg(h, src_p, dst_p)
  return _head(h, a, c3_W1, c3_b1, c3_W2, c3_b2,
               lin1_W, lin1_b, lin2_W, lin2_b)


# speedup vs baseline: 3.0243x; 1.1741x over previous
import functools

import jax
import jax.numpy as jnp
from jax import lax
from jax.experimental import pallas as pl
from jax.experimental.pallas import tpu as pltpu
from jax.experimental.pallas import tpu_sc as plsc

N = 10000
E = 320000
D = 128

NC = 2
NS = 16
NW = NC * NS

CH = 128
NCHUNK = 80
EPT = CH * NCHUNK
E_PAD = EPT * NW
ACC_ROWS = 10240
ZROWS = ACC_ROWS // NS


def _sc_agg_body(x_hbm, src_hbm, dst_hbm, out_hbm,
                 dst_v, sbuf0, sbuf1, rows0, rows1, acc,
                 isem0, isem1, gsem0, gsem1):
  core = lax.axis_index("c")
  sid = lax.axis_index("s")
  tid = sid * NC + core
  sbufs = (sbuf0, sbuf1)
  rowbufs = (rows0, rows1)
  isems = (isem0, isem1)
  gsems = (gsem0, gsem1)

  def _zero_body(k, _):
    r = k // 8
    c = (k % 8) * 16
    rows0[r, pl.ds(c, 16)] = jnp.zeros((16,), jnp.float32)
    return 0
  lax.fori_loop(0, CH * 8, _zero_body, 0)
  for z in range(ZROWS // CH):
    pltpu.sync_copy(rows0, acc.at[pl.ds(sid * ZROWS + z * CH, CH)])
  plsc.subcore_barrier()

  pltpu.sync_copy(dst_hbm.at[pl.ds(tid * NCHUNK, NCHUNK)], dst_v)

  ebase = tid * EPT

  def _idx_start(b, j):
    pltpu.async_copy(src_hbm.at[pl.ds(ebase + j * CH, CH)],
                     sbufs[b], isems[b])

  def _gather_start(b):
    pltpu.async_copy(x_hbm.at[sbufs[b]], rowbufs[b], gsems[b])

  def _gather_wait(b):
    pltpu.make_async_copy(x_hbm.at[sbufs[b]], rowbufs[b], gsems[b]).wait()

  def _idx_wait(b, j):
    pltpu.make_async_copy(src_hbm.at[pl.ds(ebase + j * CH, CH)],
                          sbufs[b], isems[b]).wait()

  for b in range(2):
    _idx_start(b, b)
    _idx_wait(b, b)
    _gather_start(b)

  def _stage(b, j, prefetch):
    _gather_wait(b)
    if prefetch:
      _idx_start(b, j + 2)
    pltpu.sync_copy(rowbufs[b], acc.at[dst_v.at[j]], add=True)
    if prefetch:
      _idx_wait(b, j + 2)
      _gather_start(b)

  def _pipe_body(t, _):
    _stage(0, 2 * t, True)
    _stage(1, 2 * t + 1, True)
    return 0
  lax.fori_loop(0, NCHUNK // 2 - 1, _pipe_body, 0)
  _stage(0, NCHUNK - 2, False)
  _stage(1, NCHUNK - 1, False)

  plsc.subcore_barrier()
  rpt = 624
  pltpu.sync_copy(acc.at[pl.ds(sid * rpt, rpt)],
                  out_hbm.at[pl.ds(core * N + sid * rpt, rpt)])

  @pl.when(sid == NS - 1)
  def _tail():
    tail0 = NS * rpt
    pltpu.sync_copy(acc.at[pl.ds(tail0, N - tail0)],
                    out_hbm.at[pl.ds(core * N + tail0, N - tail0)])


@functools.cache
def _sc_agg():
  return functools.partial(
      pl.kernel,
      out_type=jax.ShapeDtypeStruct((NC * N, D), jnp.float32),
      mesh=plsc.VectorSubcoreMesh(
          core_axis_name="c", subcore_axis_name="s", num_cores=NC,
          num_subcores=NS),
      scratch_types=[
          pltpu.VMEM((NCHUNK, CH), jnp.int32),
          pltpu.VMEM((CH,), jnp.int32),
          pltpu.VMEM((CH,), jnp.int32),
          pltpu.VMEM((CH, D), jnp.float32),
          pltpu.VMEM((CH, D), jnp.float32),
          pltpu.VMEM_SHARED((ACC_ROWS, D), jnp.float32),
          pltpu.SemaphoreType.DMA,
          pltpu.SemaphoreType.DMA,
          pltpu.SemaphoreType.DMA,
          pltpu.SemaphoreType.DMA,
      ],
  )(_sc_agg_body)


def _agg(h, src_p, dst_p):
  return _sc_agg()(h, src_p, dst_p)


ROWS_BLK = 1000
GRID = N // ROWS_BLK


def _mlp_body(relu_out, x_ref, a0_ref, a1_ref, w1_ref, b1_ref, w2_ref,
              b2_ref, o_ref):
  h = x_ref[...] + a0_ref[...] + a1_ref[...]
  h = jnp.maximum(
      jnp.dot(h, w1_ref[...], preferred_element_type=jnp.float32)
      + b1_ref[...], 0.0)
  h = jnp.dot(h, w2_ref[...], preferred_element_type=jnp.float32) + b2_ref[...]
  if relu_out:
    h = jnp.maximum(h, 0.0)
  o_ref[...] = h


def _row_spec(shift=0):
  return pl.BlockSpec((ROWS_BLK, D), lambda i: (i + shift, 0))


def _w_spec():
  return pl.BlockSpec((D, D), lambda i: (0, 0))


def _b_spec():
  return pl.BlockSpec((1, D), lambda i: (0, 0))


def _mlp(x, agg2, w1, b1, w2, b2, relu_out):
  return pl.pallas_call(
      functools.partial(_mlp_body, relu_out),
      grid=(GRID,),
      in_specs=[_row_spec(), _row_spec(), _row_spec(GRID),
                _w_spec(), _b_spec(), _w_spec(), _b_spec()],
      out_specs=_row_spec(),
      out_shape=jax.ShapeDtypeStruct((N, D), jnp.float32),
  )(x, agg2, agg2, w1, b1.reshape(1, D), w2, b2.reshape(1, D))


def _head_body(x_ref, a0_ref, a1_ref, w1_ref, b1_ref, w2_ref, b2_ref,
               l1w_ref, l1b_ref, l2w_ref, l2b_ref, o_ref):
  h = x_ref[...] + a0_ref[...] + a1_ref[...]
  h = jnp.maximum(
      jnp.dot(h, w1_ref[...], preferred_element_type=jnp.float32)
      + b1_ref[...], 0.0)
  h = jnp.dot(h, w2_ref[...], preferred_element_type=jnp.float32) + b2_ref[...]
  h = jnp.maximum(
      jnp.dot(h, l1w_ref[...], preferred_element_type=jnp.float32)
      + l1b_ref[...], 0.0)
  z = jnp.dot(h, l2w_ref[...], preferred_element_type=jnp.float32) + l2b_ref[...]
  m = jnp.max(z, axis=1, keepdims=True)
  e = z - m
  o_ref[...] = e - jnp.log(jnp.sum(jnp.exp(e), axis=1, keepdims=True))


def _head(x, agg2, w1, b1, w2, b2, l1w, l1b, l2w, l2b):
  return pl.pallas_call(
      _head_body,
      grid=(GRID,),
      in_specs=[_row_spec(), _row_spec(), _row_spec(GRID),
                _w_spec(), _b_spec(), _w_spec(), _b_spec(),
                _w_spec(), _b_spec(), _w_spec(), _b_spec()],
      out_specs=_row_spec(),
      out_shape=jax.ShapeDtypeStruct((N, D), jnp.float32),
  )(x, agg2, agg2, w1, b1.reshape(1, D), w2, b2.reshape(1, D),
    l1w, l1b.reshape(1, D), l2w, l2b.reshape(1, D))


def kernel(x, edge_index, batch, pool,
           c1_W1, c1_b1, c1_W2, c1_b2,
           c2_W1, c2_b1, c2_W2, c2_b2,
           c3_W1, c3_b1, c3_W2, c3_b2,
           lin1_W, lin1_b, lin2_W, lin2_b):
  src = edge_index[0]
  dst = edge_index[1]
  pad = E_PAD - E
  src_p = jnp.concatenate([src, jnp.zeros((pad,), jnp.int32)])
  dst_p = jnp.concatenate([dst, jnp.full((pad,), N, jnp.int32)])
  dst_p = dst_p.reshape(NW * NCHUNK, CH)

  a = _agg(x, src_p, dst_p)
  h = _mlp(x, a, c1_W1, c1_b1, c1_W2, c1_b2, relu_out=True)
  a = _agg(h, src_p, dst_p)
  h = _mlp(h, a, c2_W1, c2_b1, c2_W2, c2_b2, relu_out=True)
  a = _agg(h, src_p, dst_p)
  return _head(h, a, c3_W1, c3_b1, c3_W2, c3_b2,
               lin1_W, lin1_b, lin2_W, lin2_b)


# speedup vs baseline: 3.6200x; 1.1969x over previous
import functools

import jax
import jax.numpy as jnp
from jax import lax
from jax.experimental import pallas as pl
from jax.experimental.pallas import tpu as pltpu
from jax.experimental.pallas import tpu_sc as plsc

N = 10000
E = 320000
D = 128

NC = 2
NS = 16
NW = NC * NS

CH = 128
NCH0 = 128
NCH1 = 32
NCH_PAIR = NCH0 + NCH1
E_PAD = NS * NCH_PAIR * CH
ACC_ROWS = 10112
ZROWS = ACC_ROWS // NS


def _sc_agg_body(x_hbm, src_hbm, dst_hbm, out_hbm,
                 dst_v, sbuf0, sbuf1, rows0, rows1, acc,
                 isem0, isem1, gsem0, gsem1):
  core = lax.axis_index("c")
  sid = lax.axis_index("s")
  sbufs = (sbuf0, sbuf1)
  rowbufs = (rows0, rows1)
  isems = (isem0, isem1)
  gsems = (gsem0, gsem1)

  def _zero_body(k, _):
    r = k // 8
    c = (k % 8) * 16
    rows0[r, pl.ds(c, 16)] = jnp.zeros((16,), jnp.float32)
    return 0
  lax.fori_loop(0, CH * 8, _zero_body, 0)
  zbase = sid * ZROWS
  for z in range(ZROWS // CH):
    pltpu.sync_copy(rows0, acc.at[pl.ds(zbase + z * CH, CH)])
  zrem = ZROWS % CH
  if zrem:
    pltpu.sync_copy(rows0.at[pl.ds(0, zrem)],
                    acc.at[pl.ds(zbase + ZROWS - zrem, zrem)])
  plsc.subcore_barrier()

  def _pipe(nch, cbase):
    pltpu.sync_copy(dst_hbm.at[pl.ds(cbase, nch)], dst_v.at[pl.ds(0, nch)])
    ebase = cbase * CH

    def _idx_start(b, j):
      pltpu.async_copy(src_hbm.at[pl.ds(ebase + j * CH, CH)],
                       sbufs[b], isems[b])

    def _gather_start(b):
      pltpu.async_copy(x_hbm.at[sbufs[b]], rowbufs[b], gsems[b])

    def _gather_wait(b):
      pltpu.make_async_copy(x_hbm.at[sbufs[b]], rowbufs[b], gsems[b]).wait()

    def _idx_wait(b, j):
      pltpu.make_async_copy(src_hbm.at[pl.ds(ebase + j * CH, CH)],
                            sbufs[b], isems[b]).wait()

    for b in range(2):
      _idx_start(b, b)
      _idx_wait(b, b)
      _gather_start(b)

    def _stage(b, j, prefetch):
      _gather_wait(b)
      if prefetch:
        _idx_start(b, j + 2)
      pltpu.sync_copy(rowbufs[b], acc.at[dst_v.at[j]], add=True)
      if prefetch:
        _idx_wait(b, j + 2)
        _gather_start(b)

    def _pipe_body(t, _):
      _stage(0, 2 * t, True)
      _stage(1, 2 * t + 1, True)
      return 0
    lax.fori_loop(0, nch // 2 - 1, _pipe_body, 0)
    _stage(0, nch - 2, False)
    _stage(1, nch - 1, False)

  @pl.when(core == 0)
  def _fast_core():
    _pipe(NCH0, sid * NCH_PAIR)

  @pl.when(core == 1)
  def _slow_core():
    _pipe(NCH1, sid * NCH_PAIR + NCH0)

  plsc.subcore_barrier()
  rpt = 624
  pltpu.sync_copy(acc.at[pl.ds(sid * rpt, rpt)],
                  out_hbm.at[pl.ds(core * N + sid * rpt, rpt)])

  @pl.when(sid == NS - 1)
  def _tail():
    tail0 = NS * rpt
    pltpu.sync_copy(acc.at[pl.ds(tail0, N - tail0)],
                    out_hbm.at[pl.ds(core * N + tail0, N - tail0)])


@functools.cache
def _sc_agg():
  return functools.partial(
      pl.kernel,
      out_type=jax.ShapeDtypeStruct((NC * N, D), jnp.float32),
      mesh=plsc.VectorSubcoreMesh(
          core_axis_name="c", subcore_axis_name="s", num_cores=NC,
          num_subcores=NS),
      scratch_types=[
          pltpu.VMEM((NCH0, CH), jnp.int32),
          pltpu.VMEM((CH,), jnp.int32),
          pltpu.VMEM((CH,), jnp.int32),
          pltpu.VMEM((CH, D), jnp.float32),
          pltpu.VMEM((CH, D), jnp.float32),
          pltpu.VMEM_SHARED((ACC_ROWS, D), jnp.float32),
          pltpu.SemaphoreType.DMA,
          pltpu.SemaphoreType.DMA,
          pltpu.SemaphoreType.DMA,
          pltpu.SemaphoreType.DMA,
      ],
  )(_sc_agg_body)


def _agg(h, src_p, dst_p):
  return _sc_agg()(h, src_p, dst_p)


ROWS_BLK = 1000
GRID = N // ROWS_BLK


def _mlp_body(relu_out, x_ref, a0_ref, a1_ref, w1_ref, b1_ref, w2_ref,
              b2_ref, o_ref):
  h = x_ref[...] + a0_ref[...] + a1_ref[...]
  h = jnp.maximum(
      jnp.dot(h, w1_ref[...], preferred_element_type=jnp.float32)
      + b1_ref[...], 0.0)
  h = jnp.dot(h, w2_ref[...], preferred_element_type=jnp.float32) + b2_ref[...]
  if relu_out:
    h = jnp.maximum(h, 0.0)
  o_ref[...] = h


def _row_spec(shift=0):
  return pl.BlockSpec((ROWS_BLK, D), lambda i: (i + shift, 0))


def _w_spec():
  return pl.BlockSpec((D, D), lambda i: (0, 0))


def _b_spec():
  return pl.BlockSpec((1, D), lambda i: (0, 0))


def _mlp(x, agg2, w1, b1, w2, b2, relu_out):
  return pl.pallas_call(
      functools.partial(_mlp_body, relu_out),
      grid=(GRID,),
      in_specs=[_row_spec(), _row_spec(), _row_spec(GRID),
                _w_spec(), _b_spec(), _w_spec(), _b_spec()],
      out_specs=_row_spec(),
      out_shape=jax.ShapeDtypeStruct((N, D), jnp.float32),
  )(x, agg2, agg2, w1, b1.reshape(1, D), w2, b2.reshape(1, D))


def _head_body(x_ref, a0_ref, a1_ref, w1_ref, b1_ref, w2_ref, b2_ref,
               l1w_ref, l1b_ref, l2w_ref, l2b_ref, o_ref):
  h = x_ref[...] + a0_ref[...] + a1_ref[...]
  h = jnp.maximum(
      jnp.dot(h, w1_ref[...], preferred_element_type=jnp.float32)
      + b1_ref[...], 0.0)
  h = jnp.dot(h, w2_ref[...], preferred_element_type=jnp.float32) + b2_ref[...]
  h = jnp.maximum(
      jnp.dot(h, l1w_ref[...], preferred_element_type=jnp.float32)
      + l1b_ref[...], 0.0)
  z = jnp.dot(h, l2w_ref[...], preferred_element_type=jnp.float32) + l2b_ref[...]
  m = jnp.max(z, axis=1, keepdims=True)
  e = z - m
  o_ref[...] = e - jnp.log(jnp.sum(jnp.exp(e), axis=1, keepdims=True))


def _head(x, agg2, w1, b1, w2, b2, l1w, l1b, l2w, l2b):
  return pl.pallas_call(
      _head_body,
      grid=(GRID,),
      in_specs=[_row_spec(), _row_spec(), _row_spec(GRID),
                _w_spec(), _b_spec(), _w_spec(), _b_spec(),
                _w_spec(), _b_spec(), _w_spec(), _b_spec()],
      out_specs=_row_spec(),
      out_shape=jax.ShapeDtypeStruct((N, D), jnp.float32),
  )(x, agg2, agg2, w1, b1.reshape(1, D), w2, b2.reshape(1, D),
    l1w, l1b.reshape(1, D), l2w, l2b.reshape(1, D))


def kernel(x, edge_index, batch, pool,
           c1_W1, c1_b1, c1_W2, c1_b2,
           c2_W1, c2_b1, c2_W2, c2_b2,
           c3_W1, c3_b1, c3_W2, c3_b2,
           lin1_W, lin1_b, lin2_W, lin2_b):
  src = edge_index[0]
  dst = edge_index[1]
  pad = E_PAD - E
  src_p = jnp.concatenate([src, jnp.zeros((pad,), jnp.int32)])
  dst_p = jnp.concatenate([dst, jnp.full((pad,), N, jnp.int32)])
  dst_p = dst_p.reshape(NS * NCH_PAIR, CH)

  a = _agg(x, src_p, dst_p)
  h = _mlp(x, a, c1_W1, c1_b1, c1_W2, c1_b2, relu_out=True)
  a = _agg(h, src_p, dst_p)
  h = _mlp(h, a, c2_W1, c2_b1, c2_W2, c2_b2, relu_out=True)
  a = _agg(h, src_p, dst_p)
  return _head(h, a, c3_W1, c3_b1, c3_W2, c3_b2,
               lin1_W, lin1_b, lin2_W, lin2_b)


# speedup vs baseline: 3.6786x; 1.0162x over previous
import functools

import jax
import jax.numpy as jnp
from jax import lax
from jax.experimental import pallas as pl
from jax.experimental.pallas import tpu as pltpu
from jax.experimental.pallas import tpu_sc as plsc

N = 10000
E = 320000
D = 128

NC = 2
NS = 16
NW = NC * NS

CH = 64
NBUF = 4
NCH0 = 256
NCH1 = 64
NCH_PAIR = NCH0 + NCH1
E_PAD = NS * NCH_PAIR * CH
ACC_ROWS = 10112
ZROWS = ACC_ROWS // NS
ZCH = 128


def _sc_agg_body(x_hbm, src_hbm, dst_hbm, out_hbm,
                 sbuf0, sbuf1, sbuf2, sbuf3,
                 dbuf0, dbuf1, dbuf2, dbuf3,
                 rows0, rows1, rows2, rows3, zbuf, acc,
                 isem0, isem1, isem2, isem3,
                 dsem0, dsem1, dsem2, dsem3,
                 gsem0, gsem1, gsem2, gsem3):
  core = lax.axis_index("c")
  sid = lax.axis_index("s")
  sbufs = (sbuf0, sbuf1, sbuf2, sbuf3)
  dbufs = (dbuf0, dbuf1, dbuf2, dbuf3)
  rowbufs = (rows0, rows1, rows2, rows3)
  isems = (isem0, isem1, isem2, isem3)
  dsems = (dsem0, dsem1, dsem2, dsem3)
  gsems = (gsem0, gsem1, gsem2, gsem3)

  def _zero_body(k, _):
    r = k // 8
    c = (k % 8) * 16
    zbuf[r, pl.ds(c, 16)] = jnp.zeros((16,), jnp.float32)
    return 0
  lax.fori_loop(0, ZCH * 8, _zero_body, 0)
  zbase = sid * ZROWS
  for z in range(ZROWS // ZCH):
    pltpu.sync_copy(zbuf, acc.at[pl.ds(zbase + z * ZCH, ZCH)])
  zrem = ZROWS % ZCH
  if zrem:
    pltpu.sync_copy(zbuf.at[pl.ds(0, zrem)],
                    acc.at[pl.ds(zbase + ZROWS - zrem, zrem)])
  plsc.subcore_barrier()

  def _pipe(nch, cbase):
    ebase = cbase * CH

    def _src_start(b, j):
      pltpu.async_copy(src_hbm.at[pl.ds(ebase + j * CH, CH)],
                       sbufs[b], isems[b])

    def _src_wait(b, j):
      pltpu.make_async_copy(src_hbm.at[pl.ds(ebase + j * CH, CH)],
                            sbufs[b], isems[b]).wait()

    def _dst_start(b, j):
      pltpu.async_copy(dst_hbm.at[pl.ds(ebase + j * CH, CH)],
                       dbufs[b], dsems[b])

    def _dst_wait(b, j):
      pltpu.make_async_copy(dst_hbm.at[pl.ds(ebase + j * CH, CH)],
                            dbufs[b], dsems[b]).wait()

    def _gather_start(b):
      pltpu.async_copy(x_hbm.at[sbufs[b]], rowbufs[b], gsems[b])

    def _gather_wait(b):
      pltpu.make_async_copy(x_hbm.at[sbufs[b]], rowbufs[b], gsems[b]).wait()

    for b in range(NBUF):
      _src_start(b, b)
      _dst_start(b, b)
      _src_wait(b, b)
      _gather_start(b)

    def _stage(b, j, prefetch):
      _gather_wait(b)
      if prefetch:
        _src_start(b, j + NBUF)
      _dst_wait(b, j)
      pltpu.sync_copy(rowbufs[b], acc.at[dbufs[b]], add=True)
      if prefetch:
        _dst_start(b, j + NBUF)
        _src_wait(b, j + NBUF)
        _gather_start(b)

    def _pipe_body(t, _):
      for b in range(NBUF):
        _stage(b, NBUF * t + b, True)
      return 0
    lax.fori_loop(0, nch // NBUF - 1, _pipe_body, 0)
    for b in range(NBUF):
      _stage(b, nch - NBUF + b, False)

  @pl.when(core == 0)
  def _fast_core():
    _pipe(NCH0, sid * NCH_PAIR)

  @pl.when(core == 1)
  def _slow_core():
    _pipe(NCH1, sid * NCH_PAIR + NCH0)

  plsc.subcore_barrier()
  rpt = 624
  pltpu.sync_copy(acc.at[pl.ds(sid * rpt, rpt)],
                  out_hbm.at[pl.ds(core * N + sid * rpt, rpt)])

  @pl.when(sid == NS - 1)
  def _tail():
    tail0 = NS * rpt
    pltpu.sync_copy(acc.at[pl.ds(tail0, N - tail0)],
                    out_hbm.at[pl.ds(core * N + tail0, N - tail0)])


@functools.cache
def _sc_agg():
  return functools.partial(
      pl.kernel,
      out_type=jax.ShapeDtypeStruct((NC * N, D), jnp.float32),
      mesh=plsc.VectorSubcoreMesh(
          core_axis_name="c", subcore_axis_name="s", num_cores=NC,
          num_subcores=NS),
      scratch_types=(
          [pltpu.VMEM((CH,), jnp.int32)] * NBUF
          + [pltpu.VMEM((CH,), jnp.int32)] * NBUF
          + [pltpu.VMEM((CH, D), jnp.float32)] * NBUF
          + [pltpu.VMEM((ZCH, D), jnp.float32)]
          + [pltpu.VMEM_SHARED((ACC_ROWS, D), jnp.float32)]
          + [pltpu.SemaphoreType.DMA] * (3 * NBUF)
      ),
  )(_sc_agg_body)


def _agg(h, src_p, dst_p):
  return _sc_agg()(h, src_p, dst_p)


ROWS_BLK = 1000
GRID = N // ROWS_BLK


def _mlp_body(relu_out, x_ref, a0_ref, a1_ref, w1_ref, b1_ref, w2_ref,
              b2_ref, o_ref):
  h = x_ref[...] + a0_ref[...] + a1_ref[...]
  h = jnp.maximum(
      jnp.dot(h, w1_ref[...], preferred_element_type=jnp.float32)
      + b1_ref[...], 0.0)
  h = jnp.dot(h, w2_ref[...], preferred_element_type=jnp.float32) + b2_ref[...]
  if relu_out:
    h = jnp.maximum(h, 0.0)
  o_ref[...] = h


def _row_spec(shift=0):
  return pl.BlockSpec((ROWS_BLK, D), lambda i: (i + shift, 0))


def _w_spec():
  return pl.BlockSpec((D, D), lambda i: (0, 0))


def _b_spec():
  return pl.BlockSpec((1, D), lambda i: (0, 0))


def _mlp(x, agg2, w1, b1, w2, b2, relu_out):
  return pl.pallas_call(
      functools.partial(_mlp_body, relu_out),
      grid=(GRID,),
      in_specs=[_row_spec(), _row_spec(), _row_spec(GRID),
                _w_spec(), _b_spec(), _w_spec(), _b_spec()],
      out_specs=_row_spec(),
      out_shape=jax.ShapeDtypeStruct((N, D), jnp.float32),
  )(x, agg2, agg2, w1, b1.reshape(1, D), w2, b2.reshape(1, D))


def _head_body(x_ref, a0_ref, a1_ref, w1_ref, b1_ref, w2_ref, b2_ref,
               l1w_ref, l1b_ref, l2w_ref, l2b_ref, o_ref):
  h = x_ref[...] + a0_ref[...] + a1_ref[...]
  h = jnp.maximum(
      jnp.dot(h, w1_ref[...], preferred_element_type=jnp.float32)
      + b1_ref[...], 0.0)
  h = jnp.dot(h, w2_ref[...], preferred_element_type=jnp.float32) + b2_ref[...]
  h = jnp.maximum(
      jnp.dot(h, l1w_ref[...], preferred_element_type=jnp.float32)
      + l1b_ref[...], 0.0)
  z = jnp.dot(h, l2w_ref[...], preferred_element_type=jnp.float32) + l2b_ref[...]
  m = jnp.max(z, axis=1, keepdims=True)
  e = z - m
  o_ref[...] = e - jnp.log(jnp.sum(jnp.exp(e), axis=1, keepdims=True))


def _head(x, agg2, w1, b1, w2, b2, l1w, l1b, l2w, l2b):
  return pl.pallas_call(
      _head_body,
      grid=(GRID,),
      in_specs=[_row_spec(), _row_spec(), _row_spec(GRID),
                _w_spec(), _b_spec(), _w_spec(), _b_spec(),
                _w_spec(), _b_spec(), _w_spec(), _b_spec()],
      out_specs=_row_spec(),
      out_shape=jax.ShapeDtypeStruct((N, D), jnp.float32),
  )(x, agg2, agg2, w1, b1.reshape(1, D), w2, b2.reshape(1, D),
    l1w, l1b.reshape(1, D), l2w, l2b.reshape(1, D))


def kernel(x, edge_index, batch, pool,
           c1_W1, c1_b1, c1_W2, c1_b2,
           c2_W1, c2_b1, c2_W2, c2_b2,
           c3_W1, c3_b1, c3_W2, c3_b2,
           lin1_W, lin1_b, lin2_W, lin2_b):
  src = edge_index[0]
  dst = edge_index[1]
  pad = E_PAD - E
  src_p = jnp.concatenate([src, jnp.zeros((pad,), jnp.int32)])
  dst_p = jnp.concatenate([dst, jnp.full((pad,), N, jnp.int32)])

  a = _agg(x, src_p, dst_p)
  h = _mlp(x, a, c1_W1, c1_b1, c1_W2, c1_b2, relu_out=True)
  a = _agg(h, src_p, dst_p)
  h = _mlp(h, a, c2_W1, c2_b1, c2_W2, c2_b2, relu_out=True)
  a = _agg(h, src_p, dst_p)
  return _head(h, a, c3_W1, c3_b1, c3_W2, c3_b2,
               lin1_W, lin1_b, lin2_W, lin2_b)


# speedup vs baseline: 3.6976x; 1.0052x over previous
import functools

import jax
import jax.numpy as jnp
from jax import lax
from jax.experimental import pallas as pl
from jax.experimental.pallas import tpu as pltpu
from jax.experimental.pallas import tpu_sc as plsc

N = 10000
E = 320000
D = 128

NC = 2
NS = 16
NW = NC * NS

CH = 64
NBUF = 4
NCH0 = 240
NCH1 = 80
NCH_PAIR = NCH0 + NCH1
E_PAD = NS * NCH_PAIR * CH
ACC_ROWS = 10112
ZROWS = ACC_ROWS // NS
ZCH = 64


def _sc_agg_body(x_hbm, src_hbm, dst_hbm, out_hbm,
                 sbuf0, sbuf1, sbuf2, sbuf3,
                 dbuf0, dbuf1, dbuf2, dbuf3,
                 rows0, rows1, rows2, rows3, zbuf, acc,
                 isem0, isem1, isem2, isem3,
                 dsem0, dsem1, dsem2, dsem3,
                 gsem0, gsem1, gsem2, gsem3):
  core = lax.axis_index("c")
  sid = lax.axis_index("s")
  sbufs = (sbuf0, sbuf1, sbuf2, sbuf3)
  dbufs = (dbuf0, dbuf1, dbuf2, dbuf3)
  rowbufs = (rows0, rows1, rows2, rows3)
  isems = (isem0, isem1, isem2, isem3)
  dsems = (dsem0, dsem1, dsem2, dsem3)
  gsems = (gsem0, gsem1, gsem2, gsem3)

  def _zero_body(k, _):
    r = k // 8
    c = (k % 8) * 16
    zbuf[r, pl.ds(c, 16)] = jnp.zeros((16,), jnp.float32)
    return 0
  lax.fori_loop(0, ZCH * 8, _zero_body, 0)
  zbase = sid * ZROWS
  for z in range(ZROWS // ZCH):
    pltpu.sync_copy(zbuf, acc.at[pl.ds(zbase + z * ZCH, ZCH)])
  zrem = ZROWS % ZCH
  if zrem:
    pltpu.sync_copy(zbuf.at[pl.ds(0, zrem)],
                    acc.at[pl.ds(zbase + ZROWS - zrem, zrem)])
  plsc.subcore_barrier()

  def _pipe(nch, cbase):
    ebase = cbase * CH

    def _src_start(b, j):
      pltpu.async_copy(src_hbm.at[pl.ds(ebase + j * CH, CH)],
                       sbufs[b], isems[b])

    def _src_wait(b, j):
      pltpu.make_async_copy(src_hbm.at[pl.ds(ebase + j * CH, CH)],
                            sbufs[b], isems[b]).wait()

    def _dst_start(b, j):
      pltpu.async_copy(dst_hbm.at[pl.ds(ebase + j * CH, CH)],
                       dbufs[b], dsems[b])

    def _dst_wait(b, j):
      pltpu.make_async_copy(dst_hbm.at[pl.ds(ebase + j * CH, CH)],
                            dbufs[b], dsems[b]).wait()

    def _gather_start(b):
      pltpu.async_copy(x_hbm.at[sbufs[b]], rowbufs[b], gsems[b])

    def _gather_wait(b):
      pltpu.make_async_copy(x_hbm.at[sbufs[b]], rowbufs[b], gsems[b]).wait()

    for b in range(NBUF):
      _src_start(b, b)
      _dst_start(b, b)
      _src_wait(b, b)
      _gather_start(b)

    def _stage(b, j, prefetch):
      _gather_wait(b)
      if prefetch:
        _src_start(b, j + NBUF)
      _dst_wait(b, j)
      pltpu.sync_copy(rowbufs[b], acc.at[dbufs[b]], add=True)
      if prefetch:
        _dst_start(b, j + NBUF)
        _src_wait(b, j + NBUF)
        _gather_start(b)

    def _pipe_body(t, _):
      for b in range(NBUF):
        _stage(b, NBUF * t + b, True)
      return 0
    lax.fori_loop(0, nch // NBUF - 1, _pipe_body, 0)
    for b in range(NBUF):
      _stage(b, nch - NBUF + b, False)

  @pl.when(core == 0)
  def _fast_core():
    _pipe(NCH0, sid * NCH_PAIR)

  @pl.when(core == 1)
  def _slow_core():
    _pipe(NCH1, sid * NCH_PAIR + NCH0)

  plsc.subcore_barrier()
  rpt = 624
  pltpu.sync_copy(acc.at[pl.ds(sid * rpt, rpt)],
                  out_hbm.at[pl.ds(core * N + sid * rpt, rpt)])

  @pl.when(sid == NS - 1)
  def _tail():
    tail0 = NS * rpt
    pltpu.sync_copy(acc.at[pl.ds(tail0, N - tail0)],
                    out_hbm.at[pl.ds(core * N + tail0, N - tail0)])


@functools.cache
def _sc_agg():
  return functools.partial(
      pl.kernel,
      out_type=jax.ShapeDtypeStruct((NC * N, D), jnp.float32),
      mesh=plsc.VectorSubcoreMesh(
          core_axis_name="c", subcore_axis_name="s", num_cores=NC,
          num_subcores=NS),
      scratch_types=(
          [pltpu.VMEM((CH,), jnp.int32)] * NBUF
          + [pltpu.VMEM((CH,), jnp.int32)] * NBUF
          + [pltpu.VMEM((CH, D), jnp.float32)] * NBUF
          + [pltpu.VMEM((ZCH, D), jnp.float32)]
          + [pltpu.VMEM_SHARED((ACC_ROWS, D), jnp.float32)]
          + [pltpu.SemaphoreType.DMA] * (3 * NBUF)
      ),
  )(_sc_agg_body)


def _agg(h, src_p, dst_p):
  return _sc_agg()(h, src_p, dst_p)


ROWS_BLK = 1000
GRID = N // ROWS_BLK


def _mlp_body(relu_out, x_ref, a0_ref, a1_ref, w1_ref, b1_ref, w2_ref,
              b2_ref, o_ref):
  h = x_ref[...] + a0_ref[...] + a1_ref[...]
  h = jnp.maximum(
      jnp.dot(h, w1_ref[...], preferred_element_type=jnp.float32)
      + b1_ref[...], 0.0)
  h = jnp.dot(h, w2_ref[...], preferred_element_type=jnp.float32) + b2_ref[...]
  if relu_out:
    h = jnp.maximum(h, 0.0)
  o_ref[...] = h


def _row_spec(shift=0):
  return pl.BlockSpec((ROWS_BLK, D), lambda i: (i + shift, 0))


def _w_spec():
  return pl.BlockSpec((D, D), lambda i: (0, 0))


def _b_spec():
  return pl.BlockSpec((1, D), lambda i: (0, 0))


def _mlp(x, agg2, w1, b1, w2, b2, relu_out):
  return pl.pallas_call(
      functools.partial(_mlp_body, relu_out),
      grid=(GRID,),
      in_specs=[_row_spec(), _row_spec(), _row_spec(GRID),
                _w_spec(), _b_spec(), _w_spec(), _b_spec()],
      out_specs=_row_spec(),
      out_shape=jax.ShapeDtypeStruct((N, D), jnp.float32),
  )(x, agg2, agg2, w1, b1.reshape(1, D), w2, b2.reshape(1, D))


def _head_body(x_ref, a0_ref, a1_ref, w1_ref, b1_ref, w2_ref, b2_ref,
               l1w_ref, l1b_ref, l2w_ref, l2b_ref, o_ref):
  h = x_ref[...] + a0_ref[...] + a1_ref[...]
  h = jnp.maximum(
      jnp.dot(h, w1_ref[...], preferred_element_type=jnp.float32)
      + b1_ref[...], 0.0)
  h = jnp.dot(h, w2_ref[...], preferred_element_type=jnp.float32) + b2_ref[...]
  h = jnp.maximum(
      jnp.dot(h, l1w_ref[...], preferred_element_type=jnp.float32)
      + l1b_ref[...], 0.0)
  z = jnp.dot(h, l2w_ref[...], preferred_element_type=jnp.float32) + l2b_ref[...]
  m = jnp.max(z, axis=1, keepdims=True)
  e = z - m
  o_ref[...] = e - jnp.log(jnp.sum(jnp.exp(e), axis=1, keepdims=True))


def _head(x, agg2, w1, b1, w2, b2, l1w, l1b, l2w, l2b):
  return pl.pallas_call(
      _head_body,
      grid=(GRID,),
      in_specs=[_row_spec(), _row_spec(), _row_spec(GRID),
                _w_spec(), _b_spec(), _w_spec(), _b_spec(),
                _w_spec(), _b_spec(), _w_spec(), _b_spec()],
      out_specs=_row_spec(),
      out_shape=jax.ShapeDtypeStruct((N, D), jnp.float32),
  )(x, agg2, agg2, w1, b1.reshape(1, D), w2, b2.reshape(1, D),
    l1w, l1b.reshape(1, D), l2w, l2b.reshape(1, D))


def kernel(x, edge_index, batch, pool,
           c1_W1, c1_b1, c1_W2, c1_b2,
           c2_W1, c2_b1, c2_W2, c2_b2,
           c3_W1, c3_b1, c3_W2, c3_b2,
           lin1_W, lin1_b, lin2_W, lin2_b):
  src = edge_index[0]
  dst = edge_index[1]
  pad = E_PAD - E
  src_p = jnp.concatenate([src, jnp.zeros((pad,), jnp.int32)])
  dst_p = jnp.concatenate([dst, jnp.full((pad,), N, jnp.int32)])

  a = _agg(x, src_p, dst_p)
  h = _mlp(x, a, c1_W1, c1_b1, c1_W2, c1_b2, relu_out=True)
  a = _agg(h, src_p, dst_p)
  h = _mlp(h, a, c2_W1, c2_b1, c2_W2, c2_b2, relu_out=True)
  a = _agg(h, src_p, dst_p)
  return _head(h, a, c3_W1, c3_b1, c3_W2, c3_b2,
               lin1_W, lin1_b, lin2_W, lin2_b)


# speedup vs baseline: 3.7441x; 1.0126x over previous
import functools

import jax
import jax.numpy as jnp
from jax import lax
from jax.experimental import pallas as pl
from jax.experimental.pallas import tpu as pltpu
from jax.experimental.pallas import tpu_sc as plsc

N = 10000
E = 320000
D = 128

NC = 2
NS = 16
NW = NC * NS

CH = 32
NBUF = 8
NCH0 = 480
NCH1 = 160
NCH_PAIR = NCH0 + NCH1
E_PAD = NS * NCH_PAIR * CH
ACC_ROWS = 10112
ZROWS = ACC_ROWS // NS
ZCH = 64


def _sc_agg_body(x_hbm, src_hbm, dst_hbm, out_hbm, *scr):
  core = lax.axis_index("c")
  sid = lax.axis_index("s")
  sbufs = scr[0:NBUF]
  dbufs = scr[NBUF:2 * NBUF]
  rowbufs = scr[2 * NBUF:3 * NBUF]
  zbuf = scr[3 * NBUF]
  acc = scr[3 * NBUF + 1]
  isems = scr[3 * NBUF + 2:4 * NBUF + 2]
  dsems = scr[4 * NBUF + 2:5 * NBUF + 2]
  gsems = scr[5 * NBUF + 2:6 * NBUF + 2]

  def _zero_body(k, _):
    r = k // 8
    c = (k % 8) * 16
    zbuf[r, pl.ds(c, 16)] = jnp.zeros((16,), jnp.float32)
    return 0
  lax.fori_loop(0, ZCH * 8, _zero_body, 0)
  zbase = sid * ZROWS
  for z in range(ZROWS // ZCH):
    pltpu.sync_copy(zbuf, acc.at[pl.ds(zbase + z * ZCH, ZCH)])
  zrem = ZROWS % ZCH
  if zrem:
    pltpu.sync_copy(zbuf.at[pl.ds(0, zrem)],
                    acc.at[pl.ds(zbase + ZROWS - zrem, zrem)])
  plsc.subcore_barrier()

  def _pipe(nch, cbase):
    ebase = cbase * CH

    def _src_start(b, j):
      pltpu.async_copy(src_hbm.at[pl.ds(ebase + j * CH, CH)],
                       sbufs[b], isems[b])

    def _src_wait(b, j):
      pltpu.make_async_copy(src_hbm.at[pl.ds(ebase + j * CH, CH)],
                            sbufs[b], isems[b]).wait()

    def _dst_start(b, j):
      pltpu.async_copy(dst_hbm.at[pl.ds(ebase + j * CH, CH)],
                       dbufs[b], dsems[b])

    def _dst_wait(b, j):
      pltpu.make_async_copy(dst_hbm.at[pl.ds(ebase + j * CH, CH)],
                            dbufs[b], dsems[b]).wait()

    def _gather_start(b):
      pltpu.async_copy(x_hbm.at[sbufs[b]], rowbufs[b], gsems[b])

    def _gather_wait(b):
      pltpu.make_async_copy(x_hbm.at[sbufs[b]], rowbufs[b], gsems[b]).wait()

    for b in range(NBUF):
      _src_start(b, b)
      _dst_start(b, b)
      _src_wait(b, b)
      _gather_start(b)

    def _stage(b, j, prefetch):
      _gather_wait(b)
      if prefetch:
        _src_start(b, j + NBUF)
      _dst_wait(b, j)
      pltpu.sync_copy(rowbufs[b], acc.at[dbufs[b]], add=True)
      if prefetch:
        _dst_start(b, j + NBUF)
        _src_wait(b, j + NBUF)
        _gather_start(b)

    def _pipe_body(t, _):
      for b in range(NBUF):
        _stage(b, NBUF * t + b, True)
      return 0
    lax.fori_loop(0, nch // NBUF - 1, _pipe_body, 0)
    for b in range(NBUF):
      _stage(b, nch - NBUF + b, False)

  @pl.when(core == 0)
  def _fast_core():
    _pipe(NCH0, sid * NCH_PAIR)

  @pl.when(core == 1)
  def _slow_core():
    _pipe(NCH1, sid * NCH_PAIR + NCH0)

  plsc.subcore_barrier()
  rpt = 624
  pltpu.sync_copy(acc.at[pl.ds(sid * rpt, rpt)],
                  out_hbm.at[pl.ds(core * N + sid * rpt, rpt)])

  @pl.when(sid == NS - 1)
  def _tail():
    tail0 = NS * rpt
    pltpu.sync_copy(acc.at[pl.ds(tail0, N - tail0)],
                    out_hbm.at[pl.ds(core * N + tail0, N - tail0)])


@functools.cache
def _sc_agg():
  return functools.partial(
      pl.kernel,
      out_type=jax.ShapeDtypeStruct((NC * N, D), jnp.float32),
      mesh=plsc.VectorSubcoreMesh(
          core_axis_name="c", subcore_axis_name="s", num_cores=NC,
          num_subcores=NS),
      scratch_types=(
          [pltpu.VMEM((CH,), jnp.int32)] * NBUF
          + [pltpu.VMEM((CH,), jnp.int32)] * NBUF
          + [pltpu.VMEM((CH, D), jnp.float32)] * NBUF
          + [pltpu.VMEM((ZCH, D), jnp.float32)]
          + [pltpu.VMEM_SHARED((ACC_ROWS, D), jnp.float32)]
          + [pltpu.SemaphoreType.DMA] * (3 * NBUF)
      ),
  )(_sc_agg_body)


def _agg(h, src_p, dst_p):
  return _sc_agg()(h, src_p, dst_p)


ROWS_BLK = 1000
GRID = N // ROWS_BLK


def _mlp_body(relu_out, x_ref, a0_ref, a1_ref, w1_ref, b1_ref, w2_ref,
              b2_ref, o_ref):
  h = x_ref[...] + a0_ref[...] + a1_ref[...]
  h = jnp.maximum(
      jnp.dot(h, w1_ref[...], preferred_element_type=jnp.float32)
      + b1_ref[...], 0.0)
  h = jnp.dot(h, w2_ref[...], preferred_element_type=jnp.float32) + b2_ref[...]
  if relu_out:
    h = jnp.maximum(h, 0.0)
  o_ref[...] = h


def _row_spec(shift=0):
  return pl.BlockSpec((ROWS_BLK, D), lambda i: (i + shift, 0))


def _w_spec():
  return pl.BlockSpec((D, D), lambda i: (0, 0))


def _b_spec():
  return pl.BlockSpec((1, D), lambda i: (0, 0))


def _mlp(x, agg2, w1, b1, w2, b2, relu_out):
  return pl.pallas_call(
      functools.partial(_mlp_body, relu_out),
      grid=(GRID,),
      in_specs=[_row_spec(), _row_spec(), _row_spec(GRID),
                _w_spec(), _b_spec(), _w_spec(), _b_spec()],
      out_specs=_row_spec(),
      out_shape=jax.ShapeDtypeStruct((N, D), jnp.float32),
  )(x, agg2, agg2, w1, b1.reshape(1, D), w2, b2.reshape(1, D))


def _head_body(x_ref, a0_ref, a1_ref, w1_ref, b1_ref, w2_ref, b2_ref,
               l1w_ref, l1b_ref, l2w_ref, l2b_ref, o_ref):
  h = x_ref[...] + a0_ref[...] + a1_ref[...]
  h = jnp.maximum(
      jnp.dot(h, w1_ref[...], preferred_element_type=jnp.float32)
      + b1_ref[...], 0.0)
  h = jnp.dot(h, w2_ref[...], preferred_element_type=jnp.float32) + b2_ref[...]
  h = jnp.maximum(
      jnp.dot(h, l1w_ref[...], preferred_element_type=jnp.float32)
      + l1b_ref[...], 0.0)
  z = jnp.dot(h, l2w_ref[...], preferred_element_type=jnp.float32) + l2b_ref[...]
  m = jnp.max(z, axis=1, keepdims=True)
  e = z - m
  o_ref[...] = e - jnp.log(jnp.sum(jnp.exp(e), axis=1, keepdims=True))


def _head(x, agg2, w1, b1, w2, b2, l1w, l1b, l2w, l2b):
  return pl.pallas_call(
      _head_body,
      grid=(GRID,),
      in_specs=[_row_spec(), _row_spec(), _row_spec(GRID),
                _w_spec(), _b_spec(), _w_spec(), _b_spec(),
                _w_spec(), _b_spec(), _w_spec(), _b_spec()],
      out_specs=_row_spec(),
      out_shape=jax.ShapeDtypeStruct((N, D), jnp.float32),
  )(x, agg2, agg2, w1, b1.reshape(1, D), w2, b2.reshape(1, D),
    l1w, l1b.reshape(1, D), l2w, l2b.reshape(1, D))


def kernel(x, edge_index, batch, pool,
           c1_W1, c1_b1, c1_W2, c1_b2,
           c2_W1, c2_b1, c2_W2, c2_b2,
           c3_W1, c3_b1, c3_W2, c3_b2,
           lin1_W, lin1_b, lin2_W, lin2_b):
  src = edge_index[0]
  dst = edge_index[1]
  pad = E_PAD - E
  src_p = jnp.concatenate([src, jnp.zeros((pad,), jnp.int32)])
  dst_p = jnp.concatenate([dst, jnp.full((pad,), N, jnp.int32)])

  a = _agg(x, src_p, dst_p)
  h = _mlp(x, a, c1_W1, c1_b1, c1_W2, c1_b2, relu_out=True)
  a = _agg(h, src_p, dst_p)
  h = _mlp(h, a, c2_W1, c2_b1, c2_W2, c2_b2, relu_out=True)
  a = _agg(h, src_p, dst_p)
  return _head(h, a, c3_W1, c3_b1, c3_W2, c3_b2,
               lin1_W, lin1_b, lin2_W, lin2_b)
